# baseline (speedup 1.0000x reference)
import functools

import jax
import jax.numpy as jnp
from jax import lax
from jax.experimental import pallas as pl
from jax.experimental.pallas import tpu as pltpu
from jax.experimental.pallas import tpu_sc as plsc

_NEG = -3.0e38


def _logits_body(xs_ref, xl_ref, w_ref, tl_ref, th_ref, lg_ref):
    w = w_ref[...]
    xs = xs_ref[0]
    xl = xl_ref[0]
    dn = (((1,), (0,)), ((), ()))
    ys = jax.lax.dot_general(xs, w, dn, preferred_element_type=jnp.float32)
    yl = jax.lax.dot_general(xl, w, dn, preferred_element_type=jnp.float32)

    tl = tl_ref[...]
    th = th_ref[...]
    tln = tl * jax.lax.rsqrt(jnp.sum(tl * tl, axis=1, keepdims=True))
    thn = th * jax.lax.rsqrt(jnp.sum(th * th, axis=1, keepdims=True))

    dt = (((1,), (1,)), ((), ()))
    rs = jax.lax.rsqrt(jnp.sum(ys * ys, axis=1, keepdims=True))
    rl = jax.lax.rsqrt(jnp.sum(yl * yl, axis=1, keepdims=True))
    zs = jax.lax.dot_general(tln, ys * rs, dt, preferred_element_type=jnp.float32)
    zl = jax.lax.dot_general(thn, yl * rl, dt, preferred_element_type=jnp.float32)
    lg_ref[0] = zs + zl


def _topj_sc(lg):
    R, N = lg.shape
    nchunk = N // 16
    mesh = plsc.VectorSubcoreMesh(core_axis_name="c", subcore_axis_name="s")

    gdn = lax.GatherDimensionNumbers(
        offset_dims=(), collapsed_slice_dims=(0,), start_index_map=(0,))

    def _allreduce(x, op):
        lane = lax.iota(jnp.int32, 16)
        for sh in (1, 2, 4, 8):
            idx = jnp.bitwise_xor(lane, sh)[:, None]
            g = lax.gather(x, idx, gdn, (1,),
                           mode=lax.GatherScatterMode.PROMISE_IN_BOUNDS)
            x = op(x, g)
        return x

    @functools.partial(
        pl.kernel,
        out_type=jax.ShapeDtypeStruct((R, 16), jnp.float32),
        mesh=mesh,
        scratch_types=[
            pltpu.VMEM((N,), jnp.float32),
            pltpu.VMEM((16,), jnp.float32),
        ],
    )
    def k(lg_hbm, out_hbm, row_v, res_v):
        info = plsc.get_sparse_core_info()
        r = lax.axis_index("s") * info.num_cores + lax.axis_index("c")
        pltpu.sync_copy(lg_hbm.at[r], row_v)

        def maxbody(i, m):
            return jnp.maximum(m, row_v[pl.ds(i * 16, 16)])

        m = lax.fori_loop(0, nchunk, maxbody,
                          jnp.full((16,), _NEG, jnp.float32), unroll=8)
        zeros = jnp.zeros((16,), jnp.float32)
        s1, s5, s10, cnt = zeros, zeros, zeros, zeros
        for _ in range(10):
            v = _allreduce(m, jnp.maximum)

            def body(i, car):
                cv, nm = car
                x = row_v[pl.ds(i * 16, 16)]
                eq = x == v
                xm = jnp.where(eq, _NEG, x)
                row_v[pl.ds(i * 16, 16)] = xm
                return cv + jnp.where(eq, 1.0, 0.0), jnp.maximum(nm, xm)

            cv, m = lax.fori_loop(0, nchunk, body,
                                  (zeros, jnp.full((16,), _NEG, jnp.float32)),
                                  unroll=8)
            mult = _allreduce(cv, jnp.add)
            s1 = s1 + v * jnp.clip(1.0 - cnt, 0.0, mult)
            s5 = s5 + v * jnp.clip(5.0 - cnt, 0.0, mult)
            s10 = s10 + v * jnp.clip(10.0 - cnt, 0.0, mult)
            cnt = cnt + mult
        lane = lax.iota(jnp.int32, 16)
        res = (jnp.where(lane == 0, s1, 0.0)
               + jnp.where(lane == 1, s5 * 0.2, 0.0)
               + jnp.where(lane == 2, s10 * 0.1, 0.0))
        res_v[...] = res
        pltpu.sync_copy(res_v, out_hbm.at[r])

    return k(lg)


def _make_finish_body(B, C):
    def _finish_body(label_ref, pool_ref, probs_ref, hats_ref, loss_ref):
        acc = jnp.zeros((1, 16), jnp.float32)
        for b in range(B):
            blk = pool_ref[pl.ds(b * C, C), :]
            pmax = jnp.max(blk, axis=0, keepdims=True)
            ex = jnp.exp(blk - pmax)
            den = jnp.sum(ex, axis=0, keepdims=True)
            probs_ref[b] = ex / den
            ridx = jax.lax.broadcasted_iota(jnp.int32, (C, 16), 0)
            cand = jnp.where(blk == pmax, ridx, C)
            hats_ref[b] = jnp.min(cand, axis=0, keepdims=True)
            lse = jnp.log(den) + pmax
            lab = label_ref[b]
            sel = jnp.sum(jnp.where(ridx == lab, blk, 0.0), axis=0, keepdims=True)
            acc = acc + (lse - sel) * (1.0 / B)
        loss_ref[...] = acc
    return _finish_body


def kernel(x_s, coord_s, x_l, coord_l, label, W_proj, text_low, text_high):
    B, N, D = x_s.shape
    E = W_proj.shape[1]
    C = text_low.shape[0]
    BN = 2048
    NB = N // BN

    lg = pl.pallas_call(
        _logits_body,
        grid=(B, NB),
        in_specs=[
            pl.BlockSpec((1, BN, D), lambda b, j: (b, j, 0)),
            pl.BlockSpec((1, BN, D), lambda b, j: (b, j, 0)),
            pl.BlockSpec((D, E), lambda b, j: (0, 0)),
            pl.BlockSpec((C, E), lambda b, j: (0, 0)),
            pl.BlockSpec((C, E), lambda b, j: (0, 0)),
        ],
        out_specs=pl.BlockSpec((1, C, BN), lambda b, j: (b, 0, j)),
        out_shape=jax.ShapeDtypeStruct((B, C, N), jnp.float32),
        compiler_params=pltpu.CompilerParams(
            dimension_semantics=("arbitrary", "arbitrary"),
        ),
    )(x_s, x_l, W_proj, text_low, text_high)

    pooled = _topj_sc(lg.reshape(B * C, N))

    grid_spec = pltpu.PrefetchScalarGridSpec(
        num_scalar_prefetch=1,
        grid=(1,),
        in_specs=[pl.BlockSpec((B * C, 16), lambda i, *_: (0, 0))],
        out_specs=[
            pl.BlockSpec((B, C, 16), lambda i, *_: (0, 0, 0)),
            pl.BlockSpec((B, 1, 16), lambda i, *_: (0, 0, 0)),
            pl.BlockSpec((1, 16), lambda i, *_: (0, 0)),
        ],
    )
    probs, hats, loss = pl.pallas_call(
        _make_finish_body(B, C),
        grid_spec=grid_spec,
        out_shape=[
            jax.ShapeDtypeStruct((B, C, 16), jnp.float32),
            jax.ShapeDtypeStruct((B, 1, 16), jnp.int32),
            jax.ShapeDtypeStruct((1, 16), jnp.float32),
        ],
    )(label, pooled)

    Y_probs = jnp.transpose(probs[:, :, :3], (2, 0, 1))
    Y_hats = jnp.transpose(hats[:, 0, :3], (1, 0))
    return (Y_probs, Y_hats, loss[0, 0])


# speedup vs baseline: 1.2735x; 1.2735x over previous
import jax
import jax.numpy as jnp
from jax.experimental import pallas as pl
from jax.experimental.pallas import tpu as pltpu

_NEG = -1.0e30


def _fused_body(label_ref, xs_ref, xl_ref, w_ref, tl_ref, th_ref,
                probs_ref, hats_ref, loss_ref, lg_ref):
    b = pl.program_id(0)
    j = pl.program_id(1)
    nb = pl.num_programs(1)
    nbatch = pl.num_programs(0)

    w = w_ref[...]
    xs = xs_ref[0]
    xl = xl_ref[0]
    dn = (((1,), (0,)), ((), ()))
    ys = jax.lax.dot_general(xs, w, dn, preferred_element_type=jnp.float32)
    yl = jax.lax.dot_general(xl, w, dn, preferred_element_type=jnp.float32)

    tl = tl_ref[...]
    th = th_ref[...]
    tln = tl * jax.lax.rsqrt(jnp.sum(tl * tl, axis=1, keepdims=True))
    thn = th * jax.lax.rsqrt(jnp.sum(th * th, axis=1, keepdims=True))

    dt = (((1,), (1,)), ((), ()))
    rs = jax.lax.rsqrt(jnp.sum(ys * ys, axis=1, keepdims=True))
    rl = jax.lax.rsqrt(jnp.sum(yl * yl, axis=1, keepdims=True))
    zs = jax.lax.dot_general(tln, ys * rs, dt, preferred_element_type=jnp.float32)
    zl = jax.lax.dot_general(thn, yl * rl, dt, preferred_element_type=jnp.float32)
    lgT = zs + zl
    c = lgT.shape[0]
    bn = lgT.shape[1]
    pad = jnp.full((8 - c, bn), _NEG, jnp.float32)
    lg_ref[:, pl.ds(j * bn, bn)] = jnp.concatenate([lgT, pad], axis=0)

    @pl.when(j == nb - 1)
    def _tail():
        rem = lg_ref[...]
        cnt = jnp.zeros((8, 1), jnp.float32)
        s1 = jnp.zeros((8, 1), jnp.float32)
        s5 = jnp.zeros((8, 1), jnp.float32)
        s10 = jnp.zeros((8, 1), jnp.float32)
        for _ in range(10):
            v = jnp.max(rem, axis=1, keepdims=True)
            eq = rem == v
            m = jnp.sum(eq.astype(jnp.float32), axis=1, keepdims=True)
            s1 = s1 + v * jnp.clip(1.0 - cnt, 0.0, m)
            s5 = s5 + v * jnp.clip(5.0 - cnt, 0.0, m)
            s10 = s10 + v * jnp.clip(10.0 - cnt, 0.0, m)
            cnt = cnt + m
            rem = jnp.where(eq, -3.0e38, rem)
        p1 = s1
        p5 = s5 * (1.0 / 5.0)
        p10 = s10 * (1.0 / 10.0)
        pool = jnp.concatenate([p1, p5, p10, p1, p1, p1, p1, p1], axis=1)
        pmax = jnp.max(pool, axis=0, keepdims=True)
        ex = jnp.exp(pool - pmax)
        probs = ex / jnp.sum(ex, axis=0, keepdims=True)
        probs_ref[0] = probs
        ridx = jax.lax.broadcasted_iota(jnp.int32, (8, 8), 0)
        cand = jnp.where(pool == pmax, ridx, 8)
        hats_ref[0] = jnp.min(cand, axis=0, keepdims=True)
        m1 = jnp.max(p1, axis=0, keepdims=True)
        lse = jnp.log(jnp.sum(jnp.exp(p1 - m1), axis=0, keepdims=True)) + m1
        lab = label_ref[b]
        riota = jax.lax.broadcasted_iota(jnp.int32, (8, 1), 0)
        sel = jnp.sum(jnp.where(riota == lab, p1, 0.0), axis=0, keepdims=True)
        term = (lse - sel) / nbatch

        @pl.when(b == 0)
        def _init():
            loss_ref[...] = term

        @pl.when(b > 0)
        def _acc():
            loss_ref[...] = loss_ref[...] + term


def kernel(x_s, coord_s, x_l, coord_l, label, W_proj, text_low, text_high):
    B, N, D = x_s.shape
    E = W_proj.shape[1]
    C = text_low.shape[0]
    BN = 2048
    NB = N // BN

    grid_spec = pltpu.PrefetchScalarGridSpec(
        num_scalar_prefetch=1,
        grid=(B, NB),
        in_specs=[
            pl.BlockSpec((1, BN, D), lambda b, j, *_: (b, j, 0)),
            pl.BlockSpec((1, BN, D), lambda b, j, *_: (b, j, 0)),
            pl.BlockSpec((D, E), lambda b, j, *_: (0, 0)),
            pl.BlockSpec((C, E), lambda b, j, *_: (0, 0)),
            pl.BlockSpec((C, E), lambda b, j, *_: (0, 0)),
        ],
        out_specs=[
            pl.BlockSpec((1, 8, 8), lambda b, j, *_: (b, 0, 0)),
            pl.BlockSpec((1, 1, 8), lambda b, j, *_: (b, 0, 0)),
            pl.BlockSpec((1, 1), lambda b, j, *_: (0, 0)),
        ],
        scratch_shapes=[pltpu.VMEM((8, N), jnp.float32)],
    )
    probs, hats, loss = pl.pallas_call(
        _fused_body,
        grid_spec=grid_spec,
        out_shape=[
            jax.ShapeDtypeStruct((B, 8, 8), jnp.float32),
            jax.ShapeDtypeStruct((B, 1, 8), jnp.int32),
            jax.ShapeDtypeStruct((1, 1), jnp.float32),
        ],
        compiler_params=pltpu.CompilerParams(
            dimension_semantics=("arbitrary", "arbitrary"),
        ),
    )(label, x_s, x_l, W_proj, text_low, text_high)

    Y_probs = jnp.transpose(probs[:, :C, :3], (2, 0, 1))
    Y_hats = jnp.transpose(hats[:, 0, :3], (1, 0))
    return (Y_probs, Y_hats, loss[0, 0])
